# Initial kernel scaffold; baseline (speedup 1.0000x reference)
import jax
import jax.numpy as jnp
from jax.experimental import pallas as pl


def kernel(x, edge_index, W1, b1, W2, b2):
    raise NotImplementedError("write your pallas kernel here")


# speedup vs baseline: 182.5751x; 182.5751x over previous
import functools

import jax
import jax.numpy as jnp
from jax import lax
from jax.experimental import pallas as pl
from jax.experimental.pallas import tpu as pltpu
from jax.experimental.pallas import tpu_sc as plsc

N_NODES = 100000
N_EDGES = 6400000
HIDDEN = 128
NPAD = 100352
ROWS = N_EDGES // 128
NC = 2
NS = 16
NW = NC * NS
GROUPS = ROWS // 8
BASE_G = GROUPS // NW
EXTRA = GROUPS - BASE_G * NW
CH = 24
NCHUNK = BASE_G * 8 // CH
NSL = NPAD // NS


def _edge_pass_body(y_hbm, ei_hbm, zero_hbm, acc_hbm,
                    y_loc, src_v, dst_v, vals_v, src1_v, dst1_v, vals1_v,
                    acc_sh):
    c = lax.axis_index("c")
    s = lax.axis_index("s")
    wid = c * NS + s

    if True:
        pltpu.sync_copy(y_hbm, y_loc)
        @pl.when(s == 0)
        def _():
            pltpu.sync_copy(zero_hbm, acc_sh)
        plsc.subcore_barrier()

        row0 = (wid * BASE_G + jnp.minimum(wid, EXTRA)) * 8

        def chunk_body(ci, carry):
            r0 = row0 + ci * CH
            pltpu.sync_copy(ei_hbm.at[0, pl.ds(r0, CH)], src_v)
            pltpu.sync_copy(ei_hbm.at[1, pl.ds(r0, CH)], dst_v)
            for j in range(CH):
                for w in range(8):
                    idx = src_v[j, pl.ds(w * 16, 16)]
                    vals_v[j, pl.ds(w * 16, 16)] = plsc.load_gather(y_loc, [idx])
                pltpu.sync_copy(vals_v.at[j], acc_sh.at[dst_v.at[j]], add=True)
            return carry

        lax.fori_loop(0, NCHUNK, chunk_body, 0)

        @pl.when(wid < EXTRA)
        def _():
            r1 = row0 + BASE_G * 8
            pltpu.sync_copy(ei_hbm.at[0, pl.ds(r1, 8)], src1_v)
            pltpu.sync_copy(ei_hbm.at[1, pl.ds(r1, 8)], dst1_v)
            for j in range(8):
                for w in range(8):
                    idx = src1_v[j, pl.ds(w * 16, 16)]
                    vals1_v[j, pl.ds(w * 16, 16)] = plsc.load_gather(y_loc, [idx])
                pltpu.sync_copy(vals1_v.at[j], acc_sh.at[dst1_v.at[j]], add=True)

        plsc.subcore_barrier()
        pltpu.sync_copy(acc_sh.at[pl.ds(s * NSL, NSL)],
                        acc_hbm.at[c, pl.ds(s * NSL, NSL)])


def _edge_pass(y_flat, ei3, zeros_pad):
    mesh = plsc.VectorSubcoreMesh(core_axis_name="c", subcore_axis_name="s")
    f = pl.kernel(
        _edge_pass_body,
        out_type=jax.ShapeDtypeStruct((NC, NPAD), jnp.float32),
        mesh=mesh,
        compiler_params=pltpu.CompilerParams(needs_layout_passes=False),
        scratch_types=[
            pltpu.VMEM((NPAD,), jnp.float32),
            pltpu.VMEM((CH, 128), jnp.int32),
            pltpu.VMEM((CH, 128), jnp.int32),
            pltpu.VMEM((CH, 128), jnp.float32),
            pltpu.VMEM((8, 128), jnp.int32),
            pltpu.VMEM((8, 128), jnp.int32),
            pltpu.VMEM((8, 128), jnp.float32),
            pltpu.VMEM_SHARED((NPAD,), jnp.float32),
        ],
    )
    return f(y_flat, ei3, zeros_pad)


def _bf(v):
    return v.astype(jnp.bfloat16).astype(jnp.float32)


def _deg_body(cnt_ref, x_ref, dis_ref, y1_ref):
    deg = cnt_ref[0] + cnt_ref[1] + 1.0
    dis = lax.rsqrt(deg)
    dis_ref[...] = dis
    y1_ref[...] = _bf(x_ref[...]) * dis


def _mid_body(acc_ref, dis_ref, y1_ref, w1_ref, b1_ref, w2_ref, y2_ref):
    dis = dis_ref[...]
    s1 = dis * (acc_ref[0] + acc_ref[1] + y1_ref[...])
    hw = jnp.zeros_like(s1)
    for k in range(HIDDEN):
        w1k = _bf(w1_ref[0, k])
        h = jnp.maximum(s1 * w1k + b1_ref[0, k], 0.0)
        hw = hw + _bf(h) * _bf(w2_ref[0, k])
    y2_ref[...] = hw * dis


def _fin_body(acc_ref, dis_ref, y2_ref, b2_ref, out_ref):
    out_ref[...] = dis_ref[...] * (acc_ref[0] + acc_ref[1] + y2_ref[...]) \
        + b2_ref[0, 0]


_SMEM_SPEC = pl.BlockSpec(memory_space=pltpu.SMEM)


def kernel(x, edge_index, W1, b1, W2, b2):
    ei3 = edge_index.astype(jnp.int32).reshape(2, ROWS, 128)
    x2 = jnp.pad(x[:, 0], (0, NPAD - N_NODES)).reshape(NPAD // 128, 128)
    zeros_pad = jnp.zeros((NPAD,), jnp.float32)
    ones_pad = jnp.ones((NPAD,), jnp.float32)
    w1r = W1.reshape(1, HIDDEN)
    b1r = b1.reshape(1, HIDDEN)
    w2r = W2.reshape(1, HIDDEN)
    b2r = b2.reshape(1, 1)

    R = NPAD // 128
    f32_2d = jax.ShapeDtypeStruct((R, 128), jnp.float32)

    cnt = _edge_pass(ones_pad, ei3, zeros_pad)

    dis2, y12 = pl.pallas_call(
        _deg_body,
        out_shape=(f32_2d, f32_2d),
    )(cnt.reshape(NC, R, 128), x2)

    accA = _edge_pass(y12.reshape(NPAD), ei3, zeros_pad)

    y22 = pl.pallas_call(
        _mid_body,
        out_shape=f32_2d,
        in_specs=[pl.BlockSpec(), pl.BlockSpec(), pl.BlockSpec(),
                  _SMEM_SPEC, _SMEM_SPEC, _SMEM_SPEC],
    )(accA.reshape(NC, R, 128), dis2, y12, w1r, b1r, w2r)

    accB = _edge_pass(y22.reshape(NPAD), ei3, zeros_pad)

    out2 = pl.pallas_call(
        _fin_body,
        out_shape=f32_2d,
        in_specs=[pl.BlockSpec(), pl.BlockSpec(), pl.BlockSpec(), _SMEM_SPEC],
    )(accB.reshape(NC, R, 128), dis2, y22, b2r)

    return out2.reshape(NPAD)[:N_NODES][:, None]


# speedup vs baseline: 430.1513x; 2.3560x over previous
import functools

import jax
import jax.numpy as jnp
from jax import lax
from jax.experimental import pallas as pl
from jax.experimental.pallas import tpu as pltpu
from jax.experimental.pallas import tpu_sc as plsc

N_NODES = 100000
N_EDGES = 6400000
HIDDEN = 128
NPAD = 100352
ROWS = N_EDGES // 128
NC = 2
NS = 16
NW = NC * NS
GROUPS = ROWS // 8
BASE_G = GROUPS // NW
EXTRA = GROUPS - BASE_G * NW
CH = 24
NCHUNK = BASE_G * 8 // CH
NSL = NPAD // NS


def _edge_pass_body(gather, y_hbm, ei_hbm, zero_hbm, zrow_hbm, acc_hbm,
                    y_loc, src_v, dst_v, vals_v, src1_v, dst1_v, vals1_v,
                    acc_sh, sem_in0, sem_in1, sem_sc0, sem_sc1):
    c = lax.axis_index("c")
    s = lax.axis_index("s")
    wid = c * NS + s
    sem_in = (sem_in0, sem_in1)
    sem_sc = (sem_sc0, sem_sc1)

    if gather:
        pltpu.sync_copy(y_hbm, y_loc)
    else:
        ones16 = jnp.full((16,), 1.0, jnp.float32)
        for b in range(2):
            for j in range(CH):
                for w in range(8):
                    vals_v[b, j, pl.ds(w * 16, 16)] = ones16
        for j in range(8):
            for w in range(8):
                vals1_v[j, pl.ds(w * 16, 16)] = ones16

    @pl.when(s == 0)
    def _():
        pltpu.sync_copy(zero_hbm, acc_sh)
    plsc.subcore_barrier()

    row0 = (wid * BASE_G + jnp.minimum(wid, EXTRA)) * 8

    def stage(ci, b):
        r = row0 + ci * CH
        if gather:
            pltpu.async_copy(ei_hbm.at[0, pl.ds(r, CH)], src_v.at[b], sem_in[b])
        pltpu.async_copy(ei_hbm.at[1, pl.ds(r, CH)], dst_v.at[b], sem_in[b])

    def wait_stage(ci, b):
        r = row0 + ci * CH
        if gather:
            pltpu.make_async_copy(ei_hbm.at[0, pl.ds(r, CH)], src_v.at[b],
                                  sem_in[b]).wait()
        pltpu.make_async_copy(ei_hbm.at[1, pl.ds(r, CH)], dst_v.at[b],
                              sem_in[b]).wait()

    def gather_chunk(b):
        if gather:
            for j in range(CH):
                for w in range(8):
                    idx = src_v[b, j, pl.ds(w * 16, 16)]
                    vals_v[b, j, pl.ds(w * 16, 16)] = plsc.load_gather(y_loc, [idx])

    def fire(b):
        for j in range(CH):
            pltpu.async_copy(vals_v.at[b, j], acc_sh.at[dst_v.at[b, j]],
                             sem_sc[b], add=True)

    def drain(b):
        pltpu.make_async_copy(zrow_hbm, vals_v.at[b], sem_sc[b]).wait()

    stage(0, 0)

    def step(g, carry):
        for b in range(2):
            ci = 2 * g + b
            wait_stage(ci, b)
            gather_chunk(b)
            fire(b)
            @pl.when(ci >= 1)
            def _():
                drain(1 - b)
            stage(ci + 1, 1 - b)
        return carry

    lax.fori_loop(0, (NCHUNK - 1) // 2, step, 0)

    ci = NCHUNK - 1
    wait_stage(ci, 0)
    gather_chunk(0)
    fire(0)
    drain(1)

    @pl.when(wid < EXTRA)
    def _():
        r1 = row0 + BASE_G * 8
        if gather:
            pltpu.sync_copy(ei_hbm.at[0, pl.ds(r1, 8)], src1_v)
        pltpu.sync_copy(ei_hbm.at[1, pl.ds(r1, 8)], dst1_v)
        if gather:
            for j in range(8):
                for w in range(8):
                    idx = src1_v[j, pl.ds(w * 16, 16)]
                    vals1_v[j, pl.ds(w * 16, 16)] = plsc.load_gather(y_loc, [idx])
        for j in range(8):
            pltpu.async_copy(vals1_v.at[j], acc_sh.at[dst1_v.at[j]],
                             sem_sc1, add=True)

    drain(0)

    @pl.when(wid < EXTRA)
    def _():
        pltpu.make_async_copy(zrow_hbm.at[pl.ds(0, 8)], vals1_v, sem_sc1).wait()

    plsc.subcore_barrier()
    pltpu.sync_copy(acc_sh.at[pl.ds(s * NSL, NSL)],
                    acc_hbm.at[c, pl.ds(s * NSL, NSL)])


def _edge_pass(y_flat, ei3, zeros_pad, zrow, gather=True):
    mesh = plsc.VectorSubcoreMesh(core_axis_name="c", subcore_axis_name="s")
    f = pl.kernel(
        functools.partial(_edge_pass_body, gather),
        out_type=jax.ShapeDtypeStruct((NC, NPAD), jnp.float32),
        mesh=mesh,
        compiler_params=pltpu.CompilerParams(needs_layout_passes=False),
        scratch_types=[
            pltpu.VMEM((NPAD,), jnp.float32),
            pltpu.VMEM((2, CH, 128), jnp.int32),
            pltpu.VMEM((2, CH, 128), jnp.int32),
            pltpu.VMEM((2, CH, 128), jnp.float32),
            pltpu.VMEM((8, 128), jnp.int32),
            pltpu.VMEM((8, 128), jnp.int32),
            pltpu.VMEM((8, 128), jnp.float32),
            pltpu.VMEM_SHARED((NPAD,), jnp.float32),
            pltpu.SemaphoreType.DMA,
            pltpu.SemaphoreType.DMA,
            pltpu.SemaphoreType.DMA,
            pltpu.SemaphoreType.DMA,
        ],
    )
    return f(y_flat, ei3, zeros_pad, zrow)


def _bf(v):
    return v.astype(jnp.bfloat16).astype(jnp.float32)


def _deg_body(cnt_ref, x_ref, dis_ref, y1_ref):
    deg = cnt_ref[0] + cnt_ref[1] + 1.0
    dis = lax.rsqrt(deg)
    dis_ref[...] = dis
    y1_ref[...] = _bf(x_ref[...]) * dis


def _mid_body(acc_ref, dis_ref, y1_ref, w1_ref, b1_ref, w2_ref, y2_ref):
    dis = dis_ref[...]
    s1 = dis * (acc_ref[0] + acc_ref[1] + y1_ref[...])
    hw = jnp.zeros_like(s1)
    for k in range(HIDDEN):
        w1k = _bf(w1_ref[0, k])
        h = jnp.maximum(s1 * w1k + b1_ref[0, k], 0.0)
        hw = hw + _bf(h) * _bf(w2_ref[0, k])
    y2_ref[...] = hw * dis


def _fin_body(acc_ref, dis_ref, y2_ref, b2_ref, out_ref):
    out_ref[...] = dis_ref[...] * (acc_ref[0] + acc_ref[1] + y2_ref[...]) \
        + b2_ref[0, 0]


_SMEM_SPEC = pl.BlockSpec(memory_space=pltpu.SMEM)


def kernel(x, edge_index, W1, b1, W2, b2):
    ei3 = edge_index.astype(jnp.int32).reshape(2, ROWS, 128)
    x2 = jnp.pad(x[:, 0], (0, NPAD - N_NODES)).reshape(NPAD // 128, 128)
    zeros_pad = jnp.zeros((NPAD,), jnp.float32)
    zrow = jnp.zeros((CH, 128), jnp.float32)
    w1r = W1.reshape(1, HIDDEN)
    b1r = b1.reshape(1, HIDDEN)
    w2r = W2.reshape(1, HIDDEN)
    b2r = b2.reshape(1, 1)

    R = NPAD // 128
    f32_2d = jax.ShapeDtypeStruct((R, 128), jnp.float32)

    cnt = _edge_pass(zeros_pad, ei3, zeros_pad, zrow, gather=False)

    dis2, y12 = pl.pallas_call(
        _deg_body,
        out_shape=(f32_2d, f32_2d),
    )(cnt.reshape(NC, R, 128), x2)

    accA = _edge_pass(y12.reshape(NPAD), ei3, zeros_pad, zrow)

    y22 = pl.pallas_call(
        _mid_body,
        out_shape=f32_2d,
        in_specs=[pl.BlockSpec(), pl.BlockSpec(), pl.BlockSpec(),
                  _SMEM_SPEC, _SMEM_SPEC, _SMEM_SPEC],
    )(accA.reshape(NC, R, 128), dis2, y12, w1r, b1r, w2r)

    accB = _edge_pass(y22.reshape(NPAD), ei3, zeros_pad, zrow)

    out2 = pl.pallas_call(
        _fin_body,
        out_shape=f32_2d,
        in_specs=[pl.BlockSpec(), pl.BlockSpec(), pl.BlockSpec(), _SMEM_SPEC],
    )(accB.reshape(NC, R, 128), dis2, y22, b2r)

    return out2.reshape(NPAD)[:N_NODES][:, None]


# speedup vs baseline: 454.7672x; 1.0572x over previous
import functools

import jax
import jax.numpy as jnp
from jax import lax
from jax.experimental import pallas as pl
from jax.experimental.pallas import tpu as pltpu
from jax.experimental.pallas import tpu_sc as plsc

N_NODES = 100000
N_EDGES = 6400000
HIDDEN = 128
NPAD = 100352
ROWS = N_EDGES // 128
NC = 2
NS = 16
NW = NC * NS
GROUPS = ROWS // 8
BASE_G = GROUPS // NW
EXTRA = GROUPS - BASE_G * NW
CH = 24
NCHUNK = BASE_G * 8 // CH
NSL = NPAD // NS


def _edge_pass_body(gather, y_hbm, ei_hbm, zero_hbm, zrow_hbm, acc_hbm,
                    y_loc, src_v, dst_v, vals_v, src1_v, dst1_v, vals1_v,
                    acc_sh, sem_in0, sem_in1, sem_sc0, sem_sc1):
    c = lax.axis_index("c")
    s = lax.axis_index("s")
    wid = c * NS + s
    sem_in = (sem_in0, sem_in1)
    sem_sc = (sem_sc0, sem_sc1)

    if gather:
        pltpu.sync_copy(y_hbm, y_loc)
    else:
        ones16 = jnp.full((16,), 1.0, jnp.float32)
        for b in range(2):
            for j in range(CH):
                for w in range(8):
                    vals_v[b, j, pl.ds(w * 16, 16)] = ones16
        for j in range(8):
            for w in range(8):
                vals1_v[j, pl.ds(w * 16, 16)] = ones16

    @pl.when(s == 0)
    def _():
        pltpu.sync_copy(zero_hbm, acc_sh)
    plsc.subcore_barrier()

    row0 = (wid * BASE_G + jnp.minimum(wid, EXTRA)) * 8

    def stage(ci, b):
        r = row0 + ci * CH
        if gather:
            pltpu.async_copy(ei_hbm.at[0, pl.ds(r, CH)], src_v.at[b], sem_in[b])
        pltpu.async_copy(ei_hbm.at[1, pl.ds(r, CH)], dst_v.at[b], sem_in[b])

    def wait_stage(ci, b):
        r = row0 + ci * CH
        if gather:
            pltpu.make_async_copy(ei_hbm.at[0, pl.ds(r, CH)], src_v.at[b],
                                  sem_in[b]).wait()
        pltpu.make_async_copy(ei_hbm.at[1, pl.ds(r, CH)], dst_v.at[b],
                              sem_in[b]).wait()

    def gather_fire(b):
        for j in range(CH):
            if gather:
                for w in range(8):
                    idx = src_v[b, j, pl.ds(w * 16, 16)]
                    vals_v[b, j, pl.ds(w * 16, 16)] = plsc.load_gather(y_loc, [idx])
            pltpu.async_copy(vals_v.at[b, j], acc_sh.at[dst_v.at[b, j]],
                             sem_sc[b], add=True)

    def drain(b):
        pltpu.make_async_copy(zrow_hbm, vals_v.at[b], sem_sc[b]).wait()

    stage(0, 0)

    def step(g, carry):
        for b in range(2):
            ci = 2 * g + b
            wait_stage(ci, b)
            gather_fire(b)
            @pl.when(ci >= 1)
            def _():
                drain(1 - b)
            stage(ci + 1, 1 - b)
        return carry

    lax.fori_loop(0, (NCHUNK - 1) // 2, step, 0)

    ci = NCHUNK - 1
    wait_stage(ci, 0)
    gather_fire(0)
    drain(1)

    @pl.when(wid < EXTRA)
    def _():
        r1 = row0 + BASE_G * 8
        if gather:
            pltpu.sync_copy(ei_hbm.at[0, pl.ds(r1, 8)], src1_v)
        pltpu.sync_copy(ei_hbm.at[1, pl.ds(r1, 8)], dst1_v)
        if gather:
            for j in range(8):
                for w in range(8):
                    idx = src1_v[j, pl.ds(w * 16, 16)]
                    vals1_v[j, pl.ds(w * 16, 16)] = plsc.load_gather(y_loc, [idx])
        for j in range(8):
            pltpu.async_copy(vals1_v.at[j], acc_sh.at[dst1_v.at[j]],
                             sem_sc1, add=True)

    drain(0)

    @pl.when(wid < EXTRA)
    def _():
        pltpu.make_async_copy(zrow_hbm.at[pl.ds(0, 8)], vals1_v, sem_sc1).wait()

    plsc.subcore_barrier()
    pltpu.sync_copy(acc_sh.at[pl.ds(s * NSL, NSL)],
                    acc_hbm.at[c, pl.ds(s * NSL, NSL)])


def _edge_pass(y_flat, ei3, zeros_pad, zrow, gather=True):
    mesh = plsc.VectorSubcoreMesh(core_axis_name="c", subcore_axis_name="s")
    f = pl.kernel(
        functools.partial(_edge_pass_body, gather),
        out_type=jax.ShapeDtypeStruct((NC, NPAD), jnp.float32),
        mesh=mesh,
        compiler_params=pltpu.CompilerParams(needs_layout_passes=False, use_tc_tiling_on_sc=False),
        scratch_types=[
            pltpu.VMEM((NPAD,), jnp.float32),
            pltpu.VMEM((2, CH, 128), jnp.int32),
            pltpu.VMEM((2, CH, 128), jnp.int32),
            pltpu.VMEM((2, CH, 128), jnp.float32),
            pltpu.VMEM((8, 128), jnp.int32),
            pltpu.VMEM((8, 128), jnp.int32),
            pltpu.VMEM((8, 128), jnp.float32),
            pltpu.VMEM_SHARED((NPAD,), jnp.float32),
            pltpu.SemaphoreType.DMA,
            pltpu.SemaphoreType.DMA,
            pltpu.SemaphoreType.DMA,
            pltpu.SemaphoreType.DMA,
        ],
    )
    return f(y_flat, ei3, zeros_pad, zrow)


def _bf(v):
    return v.astype(jnp.bfloat16).astype(jnp.float32)


def _deg_body(cnt_ref, x_ref, dis_ref, y1_ref):
    deg = cnt_ref[0] + cnt_ref[1] + 1.0
    dis = lax.rsqrt(deg)
    dis_ref[...] = dis
    y1_ref[...] = _bf(x_ref[...]) * dis


def _mid_body(acc_ref, dis_ref, y1_ref, w1_ref, b1_ref, w2_ref, y2_ref):
    dis = dis_ref[...]
    s1 = dis * (acc_ref[0] + acc_ref[1] + y1_ref[...])
    hw = jnp.zeros_like(s1)
    for k in range(HIDDEN):
        w1k = _bf(w1_ref[0, k])
        h = jnp.maximum(s1 * w1k + b1_ref[0, k], 0.0)
        hw = hw + _bf(h) * _bf(w2_ref[0, k])
    y2_ref[...] = hw * dis


def _fin_body(acc_ref, dis_ref, y2_ref, b2_ref, out_ref):
    out_ref[...] = dis_ref[...] * (acc_ref[0] + acc_ref[1] + y2_ref[...]) \
        + b2_ref[0, 0]


_SMEM_SPEC = pl.BlockSpec(memory_space=pltpu.SMEM)


def kernel(x, edge_index, W1, b1, W2, b2):
    ei3 = edge_index.astype(jnp.int32).reshape(2, ROWS, 128)
    x2 = jnp.pad(x[:, 0], (0, NPAD - N_NODES)).reshape(NPAD // 128, 128)
    zeros_pad = jnp.zeros((NPAD,), jnp.float32)
    zrow = jnp.zeros((CH, 128), jnp.float32)
    w1r = W1.reshape(1, HIDDEN)
    b1r = b1.reshape(1, HIDDEN)
    w2r = W2.reshape(1, HIDDEN)
    b2r = b2.reshape(1, 1)

    R = NPAD // 128
    f32_2d = jax.ShapeDtypeStruct((R, 128), jnp.float32)

    cnt = _edge_pass(zeros_pad, ei3, zeros_pad, zrow, gather=False)

    dis2, y12 = pl.pallas_call(
        _deg_body,
        out_shape=(f32_2d, f32_2d),
    )(cnt.reshape(NC, R, 128), x2)

    accA = _edge_pass(y12.reshape(NPAD), ei3, zeros_pad, zrow)

    y22 = pl.pallas_call(
        _mid_body,
        out_shape=f32_2d,
        in_specs=[pl.BlockSpec(), pl.BlockSpec(), pl.BlockSpec(),
                  _SMEM_SPEC, _SMEM_SPEC, _SMEM_SPEC],
    )(accA.reshape(NC, R, 128), dis2, y12, w1r, b1r, w2r)

    accB = _edge_pass(y22.reshape(NPAD), ei3, zeros_pad, zrow)

    out2 = pl.pallas_call(
        _fin_body,
        out_shape=f32_2d,
        in_specs=[pl.BlockSpec(), pl.BlockSpec(), pl.BlockSpec(), _SMEM_SPEC],
    )(accB.reshape(NC, R, 128), dis2, y22, b2r)

    return out2.reshape(NPAD)[:N_NODES][:, None]


# speedup vs baseline: 491.6076x; 1.0810x over previous
import functools

import jax
import jax.numpy as jnp
from jax import lax
from jax.experimental import pallas as pl
from jax.experimental.pallas import tpu as pltpu
from jax.experimental.pallas import tpu_sc as plsc

N_NODES = 100000
N_EDGES = 6400000
HIDDEN = 128
NPAD = 100352
ROWS = N_EDGES // 128
NC = 2
NS = 16
NW = NC * NS
GROUPS = ROWS // 8
BASE_G = GROUPS // NW
EXTRA = GROUPS - BASE_G * NW
CH = 24
NCHUNK = BASE_G * 8 // CH
NSL = NPAD // NS


def _edge_pass_body(gather, y_hbm, ei_hbm, zero_hbm, zrow_hbm, acc_hbm,
                    y_loc, sd_v, vals_v, sd1_v, vals1_v,
                    acc_sh, sem_in0, sem_in1, sem_sc0, sem_sc1):
    c = lax.axis_index("c")
    s = lax.axis_index("s")
    wid = c * NS + s
    sem_in = (sem_in0, sem_in1)
    sem_sc = (sem_sc0, sem_sc1)

    if gather:
        pltpu.sync_copy(y_hbm, y_loc)
    else:
        ones16 = jnp.full((16,), 1.0, jnp.float32)
        for b in range(2):
            for j in range(CH):
                for w in range(8):
                    vals_v[b, j, pl.ds(w * 16, 16)] = ones16
        for j in range(8):
            for w in range(8):
                vals1_v[j, pl.ds(w * 16, 16)] = ones16

    @pl.when(s == 0)
    def _():
        pltpu.sync_copy(zero_hbm, acc_sh)
    plsc.subcore_barrier()

    row0 = (wid * BASE_G + jnp.minimum(wid, EXTRA)) * 8

    def stage(ci, b):
        r = row0 + ci * CH
        pltpu.async_copy(ei_hbm.at[pl.ds(2 * r, 2 * CH)], sd_v.at[b], sem_in[b])

    def wait_stage(ci, b):
        r = row0 + ci * CH
        pltpu.make_async_copy(ei_hbm.at[pl.ds(2 * r, 2 * CH)], sd_v.at[b],
                              sem_in[b]).wait()

    def gather_fire(b):
        for j in range(CH):
            if gather:
                for w in range(8):
                    idx = sd_v[b, 2 * j, pl.ds(w * 16, 16)]
                    vals_v[b, j, pl.ds(w * 16, 16)] = plsc.load_gather(y_loc, [idx])
            pltpu.async_copy(vals_v.at[b, j], acc_sh.at[sd_v.at[b, 2 * j + 1]],
                             sem_sc[b], add=True)

    def drain(b):
        pltpu.make_async_copy(zrow_hbm, vals_v.at[b], sem_sc[b]).wait()

    stage(0, 0)

    def step(g, carry):
        for b in range(2):
            ci = 2 * g + b
            wait_stage(ci, b)
            gather_fire(b)
            @pl.when(ci >= 1)
            def _():
                drain(1 - b)
            stage(ci + 1, 1 - b)
        return carry

    lax.fori_loop(0, (NCHUNK - 1) // 2, step, 0)

    ci = NCHUNK - 1
    wait_stage(ci, 0)
    gather_fire(0)
    drain(1)

    @pl.when(wid < EXTRA)
    def _():
        r1 = row0 + BASE_G * 8
        pltpu.sync_copy(ei_hbm.at[pl.ds(2 * r1, 16)], sd1_v)
        for j in range(8):
            if gather:
                for w in range(8):
                    idx = sd1_v[2 * j, pl.ds(w * 16, 16)]
                    vals1_v[j, pl.ds(w * 16, 16)] = plsc.load_gather(y_loc, [idx])
            pltpu.async_copy(vals1_v.at[j], acc_sh.at[sd1_v.at[2 * j + 1]],
                             sem_sc1, add=True)

    drain(0)

    @pl.when(wid < EXTRA)
    def _():
        pltpu.make_async_copy(zrow_hbm.at[pl.ds(0, 8)], vals1_v, sem_sc1).wait()

    plsc.subcore_barrier()
    pltpu.sync_copy(acc_sh.at[pl.ds(s * NSL, NSL)],
                    acc_hbm.at[c, pl.ds(s * NSL, NSL)])


def _edge_pass(y_flat, ei3, zeros_pad, zrow, gather=True):
    mesh = plsc.VectorSubcoreMesh(core_axis_name="c", subcore_axis_name="s")
    f = pl.kernel(
        functools.partial(_edge_pass_body, gather),
        out_type=jax.ShapeDtypeStruct((NC, NPAD), jnp.float32),
        mesh=mesh,
        compiler_params=pltpu.CompilerParams(needs_layout_passes=False, use_tc_tiling_on_sc=False),
        scratch_types=[
            pltpu.VMEM((NPAD,), jnp.float32),
            pltpu.VMEM((2, 2 * CH, 128), jnp.int32),
            pltpu.VMEM((2, CH, 128), jnp.float32),
            pltpu.VMEM((16, 128), jnp.int32),
            pltpu.VMEM((8, 128), jnp.float32),
            pltpu.VMEM_SHARED((NPAD,), jnp.float32),
            pltpu.SemaphoreType.DMA,
            pltpu.SemaphoreType.DMA,
            pltpu.SemaphoreType.DMA,
            pltpu.SemaphoreType.DMA,
        ],
    )
    return f(y_flat, ei3, zeros_pad, zrow)


def _bf(v):
    return v.astype(jnp.bfloat16).astype(jnp.float32)


def _deg_body(cnt_ref, x_ref, dis_ref, y1_ref):
    deg = cnt_ref[0] + cnt_ref[1] + 1.0
    dis = lax.rsqrt(deg)
    dis_ref[...] = dis
    y1_ref[...] = _bf(x_ref[...]) * dis


def _mid_body(acc_ref, dis_ref, y1_ref, w1_ref, b1_ref, w2_ref, y2_ref):
    dis = dis_ref[...]
    s1 = dis * (acc_ref[0] + acc_ref[1] + y1_ref[...])
    hw = jnp.zeros_like(s1)
    for k in range(HIDDEN):
        w1k = _bf(w1_ref[0, k])
        h = jnp.maximum(s1 * w1k + b1_ref[0, k], 0.0)
        hw = hw + _bf(h) * _bf(w2_ref[0, k])
    y2_ref[...] = hw * dis


def _fin_body(acc_ref, dis_ref, y2_ref, b2_ref, out_ref):
    out_ref[...] = dis_ref[...] * (acc_ref[0] + acc_ref[1] + y2_ref[...]) \
        + b2_ref[0, 0]


_SMEM_SPEC = pl.BlockSpec(memory_space=pltpu.SMEM)


def kernel(x, edge_index, W1, b1, W2, b2):
    ei3 = edge_index.astype(jnp.int32).reshape(2, ROWS, 128) \
        .transpose(1, 0, 2).reshape(2 * ROWS, 128)
    x2 = jnp.pad(x[:, 0], (0, NPAD - N_NODES)).reshape(NPAD // 128, 128)
    zeros_pad = jnp.zeros((NPAD,), jnp.float32)
    zrow = jnp.zeros((CH, 128), jnp.float32)
    w1r = W1.reshape(1, HIDDEN)
    b1r = b1.reshape(1, HIDDEN)
    w2r = W2.reshape(1, HIDDEN)
    b2r = b2.reshape(1, 1)

    R = NPAD // 128
    f32_2d = jax.ShapeDtypeStruct((R, 128), jnp.float32)

    cnt = _edge_pass(zeros_pad, ei3, zeros_pad, zrow, gather=False)

    dis2, y12 = pl.pallas_call(
        _deg_body,
        out_shape=(f32_2d, f32_2d),
    )(cnt.reshape(NC, R, 128), x2)

    accA = _edge_pass(y12.reshape(NPAD), ei3, zeros_pad, zrow)

    y22 = pl.pallas_call(
        _mid_body,
        out_shape=f32_2d,
        in_specs=[pl.BlockSpec(), pl.BlockSpec(), pl.BlockSpec(),
                  _SMEM_SPEC, _SMEM_SPEC, _SMEM_SPEC],
    )(accA.reshape(NC, R, 128), dis2, y12, w1r, b1r, w2r)

    accB = _edge_pass(y22.reshape(NPAD), ei3, zeros_pad, zrow)

    out2 = pl.pallas_call(
        _fin_body,
        out_shape=f32_2d,
        in_specs=[pl.BlockSpec(), pl.BlockSpec(), pl.BlockSpec(), _SMEM_SPEC],
    )(accB.reshape(NC, R, 128), dis2, y22, b2r)

    return out2.reshape(NPAD)[:N_NODES][:, None]


# speedup vs baseline: 493.1906x; 1.0032x over previous
import functools

import jax
import jax.numpy as jnp
from jax import lax
from jax.experimental import pallas as pl
from jax.experimental.pallas import tpu as pltpu
from jax.experimental.pallas import tpu_sc as plsc

N_NODES = 100000
N_EDGES = 6400000
HIDDEN = 128
NPAD = 100352
ROWS = N_EDGES // 128
NC = 2
NS = 16
NW = NC * NS
GROUPS = ROWS // 8
BASE_G = GROUPS // NW
EXTRA = GROUPS - BASE_G * NW
CH = 24
NCHUNK = BASE_G * 8 // CH
NSL = NPAD // NS


def _edge_pass_body(gather, y_hbm, ei_hbm, zero_hbm, zrow_hbm, acc_hbm,
                    y_loc, sd_v, vals_v, sd1_v, vals1_v,
                    acc_sh, sem_in0, sem_in1, sem_sc0, sem_sc1):
    c = lax.axis_index("c")
    s = lax.axis_index("s")
    wid = c * NS + s
    sem_in = (sem_in0, sem_in1)
    sem_sc = (sem_sc0, sem_sc1)

    if gather:
        pltpu.sync_copy(y_hbm, y_loc)
    else:
        ones16 = jnp.full((16,), 1.0, jnp.float32)
        for b in range(2):
            for j in range(CH):
                for w in range(8):
                    vals_v[b, j, pl.ds(w * 16, 16)] = ones16
        for j in range(8):
            for w in range(8):
                vals1_v[j, pl.ds(w * 16, 16)] = ones16

    @pl.when(s == 0)
    def _():
        pltpu.sync_copy(zero_hbm, acc_sh)
    plsc.subcore_barrier()

    row0 = (wid * BASE_G + jnp.minimum(wid, EXTRA)) * 8

    def stage(ci, b):
        r = row0 + ci * CH
        pltpu.async_copy(ei_hbm.at[pl.ds(2 * r, 2 * CH)], sd_v.at[b], sem_in[b])

    def wait_stage(ci, b):
        r = row0 + ci * CH
        pltpu.make_async_copy(ei_hbm.at[pl.ds(2 * r, 2 * CH)], sd_v.at[b],
                              sem_in[b]).wait()

    def gather_fire(b):
        for j in range(CH):
            if gather:
                for w in range(8):
                    idx = sd_v[b, 2 * j, pl.ds(w * 16, 16)]
                    vals_v[b, j, pl.ds(w * 16, 16)] = plsc.load_gather(y_loc, [idx])
            pltpu.async_copy(vals_v.at[b, j], acc_sh.at[sd_v.at[b, 2 * j + 1]],
                             sem_sc[b], add=True)

    def drain(b):
        pltpu.make_async_copy(zrow_hbm, vals_v.at[b], sem_sc[b]).wait()

    stage(0, 0)

    def step(g, carry):
        for b in range(2):
            ci = 2 * g + b
            wait_stage(ci, b)
            gather_fire(b)
            @pl.when(ci >= 1)
            def _():
                drain(1 - b)
            stage(ci + 1, 1 - b)
        return carry

    lax.fori_loop(0, (NCHUNK - 1) // 2, step, 0)

    ci = NCHUNK - 1
    wait_stage(ci, 0)
    gather_fire(0)
    drain(1)

    @pl.when(wid < EXTRA)
    def _():
        r1 = row0 + BASE_G * 8
        pltpu.sync_copy(ei_hbm.at[pl.ds(2 * r1, 16)], sd1_v)
        for j in range(8):
            if gather:
                for w in range(8):
                    idx = sd1_v[2 * j, pl.ds(w * 16, 16)]
                    vals1_v[j, pl.ds(w * 16, 16)] = plsc.load_gather(y_loc, [idx])
            pltpu.async_copy(vals1_v.at[j], acc_sh.at[sd1_v.at[2 * j + 1]],
                             sem_sc1, add=True)

    drain(0)

    @pl.when(wid < EXTRA)
    def _():
        pltpu.make_async_copy(zrow_hbm.at[pl.ds(0, 8)], vals1_v, sem_sc1).wait()

    plsc.subcore_barrier()
    pltpu.sync_copy(acc_sh.at[pl.ds(s * NSL, NSL)],
                    acc_hbm.at[c, pl.ds(s * NSL, NSL)])


def _edge_pass(y_flat, ei3, zeros_pad, zrow, gather=True):
    mesh = plsc.VectorSubcoreMesh(core_axis_name="c", subcore_axis_name="s")
    f = pl.kernel(
        functools.partial(_edge_pass_body, gather),
        out_type=jax.ShapeDtypeStruct((NC, NPAD), jnp.float32),
        mesh=mesh,
        compiler_params=pltpu.CompilerParams(needs_layout_passes=False, use_tc_tiling_on_sc=False),
        scratch_types=[
            pltpu.VMEM((NPAD,), jnp.float32),
            pltpu.VMEM((2, 2 * CH, 128), jnp.int32),
            pltpu.VMEM((2, CH, 128), jnp.float32),
            pltpu.VMEM((16, 128), jnp.int32),
            pltpu.VMEM((8, 128), jnp.float32),
            pltpu.VMEM_SHARED((NPAD,), jnp.float32),
            pltpu.SemaphoreType.DMA,
            pltpu.SemaphoreType.DMA,
            pltpu.SemaphoreType.DMA,
            pltpu.SemaphoreType.DMA,
        ],
    )
    return f(y_flat, ei3, zeros_pad, zrow)


def _bf(v):
    return v.astype(jnp.bfloat16).astype(jnp.float32)


def _deg_body(cnt_ref, x_ref, dis_ref, y1_ref):
    deg = cnt_ref[0] + cnt_ref[1] + 1.0
    dis = lax.rsqrt(deg)
    dis_ref[...] = dis
    y1_ref[...] = x_ref[...] * dis


def _mid_body(acc_ref, dis_ref, y1_ref, w1_ref, b1_ref, w2_ref, y2_ref):
    dis = dis_ref[...]
    s1 = dis * (acc_ref[0] + acc_ref[1] + y1_ref[...])
    hw = jnp.zeros_like(s1)
    for k in range(HIDDEN):
        h = jnp.maximum(s1 * w1_ref[0, k] + b1_ref[0, k], 0.0)
        hw = hw + _bf(h) * _bf(w2_ref[0, k])
    y2_ref[...] = hw * dis


def _fin_body(acc_ref, dis_ref, y2_ref, b2_ref, out_ref):
    out_ref[...] = dis_ref[...] * (acc_ref[0] + acc_ref[1] + y2_ref[...]) \
        + b2_ref[0, 0]


_SMEM_SPEC = pl.BlockSpec(memory_space=pltpu.SMEM)


def kernel(x, edge_index, W1, b1, W2, b2):
    ei3 = edge_index.astype(jnp.int32).reshape(2, ROWS, 128) \
        .transpose(1, 0, 2).reshape(2 * ROWS, 128)
    x2 = jnp.pad(x[:, 0], (0, NPAD - N_NODES)).reshape(NPAD // 128, 128)
    zeros_pad = jnp.zeros((NPAD,), jnp.float32)
    zrow = jnp.zeros((CH, 128), jnp.float32)
    w1r = W1.reshape(1, HIDDEN)
    b1r = b1.reshape(1, HIDDEN)
    w2r = W2.reshape(1, HIDDEN)
    b2r = b2.reshape(1, 1)

    R = NPAD // 128
    f32_2d = jax.ShapeDtypeStruct((R, 128), jnp.float32)

    cnt = _edge_pass(zeros_pad, ei3, zeros_pad, zrow, gather=False)

    dis2, y12 = pl.pallas_call(
        _deg_body,
        out_shape=(f32_2d, f32_2d),
    )(cnt.reshape(NC, R, 128), x2)

    accA = _edge_pass(y12.reshape(NPAD), ei3, zeros_pad, zrow)

    y22 = pl.pallas_call(
        _mid_body,
        out_shape=f32_2d,
        in_specs=[pl.BlockSpec(), pl.BlockSpec(), pl.BlockSpec(),
                  _SMEM_SPEC, _SMEM_SPEC, _SMEM_SPEC],
    )(accA.reshape(NC, R, 128), dis2, y12, w1r, b1r, w2r)

    accB = _edge_pass(y22.reshape(NPAD), ei3, zeros_pad, zrow)

    out2 = pl.pallas_call(
        _fin_body,
        out_shape=f32_2d,
        in_specs=[pl.BlockSpec(), pl.BlockSpec(), pl.BlockSpec(), _SMEM_SPEC],
    )(accB.reshape(NC, R, 128), dis2, y22, b2r)

    return out2.reshape(NPAD)[:N_NODES][:, None]


# speedup vs baseline: 498.6878x; 1.0111x over previous
import functools

import jax
import jax.numpy as jnp
from jax import lax
from jax.experimental import pallas as pl
from jax.experimental.pallas import tpu as pltpu
from jax.experimental.pallas import tpu_sc as plsc

N_NODES = 100000
N_EDGES = 6400000
HIDDEN = 128
NPAD = 100352
ROWS = N_EDGES // 128
NC = 2
NS = 16
NW = NC * NS
GROUPS = ROWS // 8
BASE_G = GROUPS // NW
EXTRA = GROUPS - BASE_G * NW
CH = 24
NCHUNK = BASE_G * 8 // CH
NSL = NPAD // NS


def _edge_pass_body(gather, y_hbm, ei_hbm, zero_hbm, zrow_hbm, acc_hbm,
                    y_loc, sd_v, vals_v, sd1_v, vals1_v,
                    acc_sh, sem_in0, sem_in1, sem_sc0, sem_sc1, sem_y):
    c = lax.axis_index("c")
    s = lax.axis_index("s")
    wid = c * NS + s
    sem_in = (sem_in0, sem_in1)
    sem_sc = (sem_sc0, sem_sc1)

    if gather:
        pltpu.async_copy(y_hbm, y_loc, sem_y)
    else:
        ones16 = jnp.full((16,), 1.0, jnp.float32)
        for b in range(2):
            for j in range(CH):
                for w in range(8):
                    vals_v[b, j, pl.ds(w * 16, 16)] = ones16
        for j in range(8):
            for w in range(8):
                vals1_v[j, pl.ds(w * 16, 16)] = ones16

    @pl.when(s == 0)
    def _():
        pltpu.sync_copy(zero_hbm, acc_sh)
    plsc.subcore_barrier()

    row0 = (wid * BASE_G + jnp.minimum(wid, EXTRA)) * 8

    def stage(ci, b):
        r = row0 + ci * CH
        pltpu.async_copy(ei_hbm.at[pl.ds(2 * r, 2 * CH)], sd_v.at[b], sem_in[b])

    def wait_stage(ci, b):
        r = row0 + ci * CH
        pltpu.make_async_copy(ei_hbm.at[pl.ds(2 * r, 2 * CH)], sd_v.at[b],
                              sem_in[b]).wait()

    def gather_fire(b):
        for j in range(CH):
            if gather:
                for w in range(8):
                    idx = sd_v[b, 2 * j, pl.ds(w * 16, 16)]
                    vals_v[b, j, pl.ds(w * 16, 16)] = plsc.load_gather(y_loc, [idx])
            pltpu.async_copy(vals_v.at[b, j], acc_sh.at[sd_v.at[b, 2 * j + 1]],
                             sem_sc[b], add=True)

    def drain(b):
        pltpu.make_async_copy(zrow_hbm, vals_v.at[b], sem_sc[b]).wait()

    stage(0, 0)
    if gather:
        pltpu.make_async_copy(y_hbm, y_loc, sem_y).wait()

    def step(g, carry):
        for b in range(2):
            ci = 2 * g + b
            wait_stage(ci, b)
            gather_fire(b)
            @pl.when(ci >= 1)
            def _():
                drain(1 - b)
            stage(ci + 1, 1 - b)
        return carry

    lax.fori_loop(0, (NCHUNK - 1) // 2, step, 0)

    ci = NCHUNK - 1
    wait_stage(ci, 0)
    gather_fire(0)
    drain(1)

    @pl.when(wid < EXTRA)
    def _():
        r1 = row0 + BASE_G * 8
        pltpu.sync_copy(ei_hbm.at[pl.ds(2 * r1, 16)], sd1_v)
        for j in range(8):
            if gather:
                for w in range(8):
                    idx = sd1_v[2 * j, pl.ds(w * 16, 16)]
                    vals1_v[j, pl.ds(w * 16, 16)] = plsc.load_gather(y_loc, [idx])
            pltpu.async_copy(vals1_v.at[j], acc_sh.at[sd1_v.at[2 * j + 1]],
                             sem_sc1, add=True)

    drain(0)

    @pl.when(wid < EXTRA)
    def _():
        pltpu.make_async_copy(zrow_hbm.at[pl.ds(0, 8)], vals1_v, sem_sc1).wait()

    plsc.subcore_barrier()
    pltpu.sync_copy(acc_sh.at[pl.ds(s * NSL, NSL)],
                    acc_hbm.at[c, pl.ds(s * NSL, NSL)])


def _edge_pass(y_flat, ei3, zeros_pad, zrow, gather=True):
    mesh = plsc.VectorSubcoreMesh(core_axis_name="c", subcore_axis_name="s")
    f = pl.kernel(
        functools.partial(_edge_pass_body, gather),
        out_type=jax.ShapeDtypeStruct((NC, NPAD), jnp.float32),
        mesh=mesh,
        compiler_params=pltpu.CompilerParams(needs_layout_passes=False, use_tc_tiling_on_sc=False),
        scratch_types=[
            pltpu.VMEM((NPAD,), jnp.float32),
            pltpu.VMEM((2, 2 * CH, 128), jnp.int32),
            pltpu.VMEM((2, CH, 128), jnp.float32),
            pltpu.VMEM((16, 128), jnp.int32),
            pltpu.VMEM((8, 128), jnp.float32),
            pltpu.VMEM_SHARED((NPAD,), jnp.float32),
            pltpu.SemaphoreType.DMA,
            pltpu.SemaphoreType.DMA,
            pltpu.SemaphoreType.DMA,
            pltpu.SemaphoreType.DMA,
            pltpu.SemaphoreType.DMA,
        ],
    )
    return f(y_flat, ei3, zeros_pad, zrow)


def _bf(v):
    return v.astype(jnp.bfloat16).astype(jnp.float32)


def _deg_body(cnt_ref, x_ref, dis_ref, y1_ref):
    deg = cnt_ref[0] + cnt_ref[1] + 1.0
    dis = lax.rsqrt(deg)
    dis_ref[...] = dis
    y1_ref[...] = x_ref[...] * dis


def _mid_body(acc_ref, dis_ref, y1_ref, w1_ref, b1_ref, w2_ref, y2_ref):
    dis = dis_ref[...]
    s1 = dis * (acc_ref[0] + acc_ref[1] + y1_ref[...])
    hw = jnp.zeros_like(s1)
    for k in range(HIDDEN):
        h = jnp.maximum(s1 * w1_ref[0, k] + b1_ref[0, k], 0.0)
        hw = hw + _bf(h) * _bf(w2_ref[0, k])
    y2_ref[...] = hw * dis


def _fin_body(acc_ref, dis_ref, y2_ref, b2_ref, out_ref):
    out_ref[...] = dis_ref[...] * (acc_ref[0] + acc_ref[1] + y2_ref[...]) \
        + b2_ref[0, 0]


_SMEM_SPEC = pl.BlockSpec(memory_space=pltpu.SMEM)


def kernel(x, edge_index, W1, b1, W2, b2):
    ei3 = edge_index.astype(jnp.int32).reshape(2, ROWS, 128) \
        .transpose(1, 0, 2).reshape(2 * ROWS, 128)
    x2 = jnp.pad(x[:, 0], (0, NPAD - N_NODES)).reshape(NPAD // 128, 128)
    zeros_pad = jnp.zeros((NPAD,), jnp.float32)
    zrow = jnp.zeros((CH, 128), jnp.float32)
    w1r = W1.reshape(1, HIDDEN)
    b1r = b1.reshape(1, HIDDEN)
    w2r = W2.reshape(1, HIDDEN)
    b2r = b2.reshape(1, 1)

    R = NPAD // 128
    f32_2d = jax.ShapeDtypeStruct((R, 128), jnp.float32)

    cnt = _edge_pass(zeros_pad, ei3, zeros_pad, zrow, gather=False)

    dis2, y12 = pl.pallas_call(
        _deg_body,
        out_shape=(f32_2d, f32_2d),
    )(cnt.reshape(NC, R, 128), x2)

    accA = _edge_pass(y12.reshape(NPAD), ei3, zeros_pad, zrow)

    y22 = pl.pallas_call(
        _mid_body,
        out_shape=f32_2d,
        in_specs=[pl.BlockSpec(), pl.BlockSpec(), pl.BlockSpec(),
                  _SMEM_SPEC, _SMEM_SPEC, _SMEM_SPEC],
    )(accA.reshape(NC, R, 128), dis2, y12, w1r, b1r, w2r)

    accB = _edge_pass(y22.reshape(NPAD), ei3, zeros_pad, zrow)

    out2 = pl.pallas_call(
        _fin_body,
        out_shape=f32_2d,
        in_specs=[pl.BlockSpec(), pl.BlockSpec(), pl.BlockSpec(), _SMEM_SPEC],
    )(accB.reshape(NC, R, 128), dis2, y22, b2r)

    return out2.reshape(NPAD)[:N_NODES][:, None]


# speedup vs baseline: 617.7283x; 1.2387x over previous
import functools

import jax
import jax.numpy as jnp
from jax import lax
from jax.experimental import pallas as pl
from jax.experimental.pallas import tpu as pltpu
from jax.experimental.pallas import tpu_sc as plsc

N_NODES = 100000
N_EDGES = 6400000
HIDDEN = 128
NPAD = 100352
ROWS = N_EDGES // 128
NC = 2
NS = 16
NW = NC * NS
GROUPS = ROWS // 8
BASE_G = GROUPS // NW
EXTRA = GROUPS - BASE_G * NW
CH = 24
NCHUNK = BASE_G * 8 // CH
NSL = NPAD // NS


def _edge_pass_body(gather, y_hbm, ei_hbm, zero_hbm, zrow_hbm, acc_hbm,
                    y_loc, sd_v, vals_v, sd1_v, vals1_v,
                    acc_sh, sem_in0, sem_in1, sem_sc0, sem_sc1, sem_y):
    c = lax.axis_index("c")
    s = lax.axis_index("s")
    wid = c * NS + s
    sem_in = (sem_in0, sem_in1)
    sem_sc = (sem_sc0, sem_sc1)

    if gather:
        pltpu.async_copy(y_hbm, y_loc, sem_y)
    else:
        ones16 = jnp.full((16,), 1.0, jnp.float32)
        for b in range(2):
            for j in range(CH):
                for w in range(8):
                    vals_v[b, j, pl.ds(w * 16, 16)] = ones16
        for j in range(8):
            for w in range(8):
                vals1_v[j, pl.ds(w * 16, 16)] = ones16

    @pl.when(s == 0)
    def _():
        pltpu.sync_copy(zero_hbm, acc_sh)
    plsc.subcore_barrier()

    row0 = (wid * BASE_G + jnp.minimum(wid, EXTRA)) * 8

    def stage(ci, b):
        r = row0 + ci * CH
        pltpu.async_copy(ei_hbm.at[pl.ds(2 * r, 2 * CH)], sd_v.at[b], sem_in[b])

    def wait_stage(ci, b):
        r = row0 + ci * CH
        pltpu.make_async_copy(ei_hbm.at[pl.ds(2 * r, 2 * CH)], sd_v.at[b],
                              sem_in[b]).wait()

    def gather_fire(b):
        for j in range(CH):
            if gather:
                idxs = [sd_v[b, 2 * j, pl.ds(w * 16, 16)] for w in range(8)]
                gs = [plsc.load_gather(y_loc, [ix]) for ix in idxs]
                for w in range(8):
                    vals_v[b, j, pl.ds(w * 16, 16)] = gs[w]
            pltpu.async_copy(vals_v.at[b, j], acc_sh.at[sd_v.at[b, 2 * j + 1]],
                             sem_sc[b], add=True)

    def drain(b):
        pltpu.make_async_copy(zrow_hbm, vals_v.at[b], sem_sc[b]).wait()

    stage(0, 0)
    if gather:
        pltpu.make_async_copy(y_hbm, y_loc, sem_y).wait()

    def step(g, carry):
        for b in range(2):
            ci = 2 * g + b
            wait_stage(ci, b)
            gather_fire(b)
            @pl.when(ci >= 1)
            def _():
                drain(1 - b)
            stage(ci + 1, 1 - b)
        return carry

    lax.fori_loop(0, (NCHUNK - 1) // 2, step, 0)

    ci = NCHUNK - 1
    wait_stage(ci, 0)
    gather_fire(0)
    drain(1)

    @pl.when(wid < EXTRA)
    def _():
        r1 = row0 + BASE_G * 8
        pltpu.sync_copy(ei_hbm.at[pl.ds(2 * r1, 16)], sd1_v)
        for j in range(8):
            if gather:
                idxs = [sd1_v[2 * j, pl.ds(w * 16, 16)] for w in range(8)]
                gs = [plsc.load_gather(y_loc, [ix]) for ix in idxs]
                for w in range(8):
                    vals1_v[j, pl.ds(w * 16, 16)] = gs[w]
            pltpu.async_copy(vals1_v.at[j], acc_sh.at[sd1_v.at[2 * j + 1]],
                             sem_sc1, add=True)

    drain(0)

    @pl.when(wid < EXTRA)
    def _():
        pltpu.make_async_copy(zrow_hbm.at[pl.ds(0, 8)], vals1_v, sem_sc1).wait()

    plsc.subcore_barrier()
    pltpu.sync_copy(acc_sh.at[pl.ds(s * NSL, NSL)],
                    acc_hbm.at[c, pl.ds(s * NSL, NSL)])


def _edge_pass(y_flat, ei3, zeros_pad, zrow, gather=True):
    mesh = plsc.VectorSubcoreMesh(core_axis_name="c", subcore_axis_name="s")
    f = pl.kernel(
        functools.partial(_edge_pass_body, gather),
        out_type=jax.ShapeDtypeStruct((NC, NPAD), jnp.float32),
        mesh=mesh,
        compiler_params=pltpu.CompilerParams(needs_layout_passes=False, use_tc_tiling_on_sc=False),
        scratch_types=[
            pltpu.VMEM((NPAD,), jnp.float32),
            pltpu.VMEM((2, 2 * CH, 128), jnp.int32),
            pltpu.VMEM((2, CH, 128), jnp.float32),
            pltpu.VMEM((16, 128), jnp.int32),
            pltpu.VMEM((8, 128), jnp.float32),
            pltpu.VMEM_SHARED((NPAD,), jnp.float32),
            pltpu.SemaphoreType.DMA,
            pltpu.SemaphoreType.DMA,
            pltpu.SemaphoreType.DMA,
            pltpu.SemaphoreType.DMA,
            pltpu.SemaphoreType.DMA,
        ],
    )
    return f(y_flat, ei3, zeros_pad, zrow)


def _bf(v):
    return v.astype(jnp.bfloat16).astype(jnp.float32)


def _deg_body(cnt_ref, x_ref, dis_ref, y1_ref):
    deg = cnt_ref[0] + cnt_ref[1] + 1.0
    dis = lax.rsqrt(deg)
    dis_ref[...] = dis
    y1_ref[...] = x_ref[...] * dis


def _mid_body(acc_ref, dis_ref, y1_ref, w1_ref, b1_ref, w2_ref, y2_ref):
    dis = dis_ref[...]
    s1 = dis * (acc_ref[0] + acc_ref[1] + y1_ref[...])
    hw = jnp.zeros_like(s1)
    for k in range(HIDDEN):
        h = jnp.maximum(s1 * w1_ref[0, k] + b1_ref[0, k], 0.0)
        hw = hw + _bf(h) * _bf(w2_ref[0, k])
    y2_ref[...] = hw * dis


def _fin_body(acc_ref, dis_ref, y2_ref, b2_ref, out_ref):
    out_ref[...] = dis_ref[...] * (acc_ref[0] + acc_ref[1] + y2_ref[...]) \
        + b2_ref[0, 0]


_SMEM_SPEC = pl.BlockSpec(memory_space=pltpu.SMEM)


def kernel(x, edge_index, W1, b1, W2, b2):
    ei3 = edge_index.astype(jnp.int32).reshape(2, ROWS, 128) \
        .transpose(1, 0, 2).reshape(2 * ROWS, 128)
    x2 = jnp.pad(x[:, 0], (0, NPAD - N_NODES)).reshape(NPAD // 128, 128)
    zeros_pad = jnp.zeros((NPAD,), jnp.float32)
    zrow = jnp.zeros((CH, 128), jnp.float32)
    w1r = W1.reshape(1, HIDDEN)
    b1r = b1.reshape(1, HIDDEN)
    w2r = W2.reshape(1, HIDDEN)
    b2r = b2.reshape(1, 1)

    R = NPAD // 128
    f32_2d = jax.ShapeDtypeStruct((R, 128), jnp.float32)

    cnt = _edge_pass(zeros_pad, ei3, zeros_pad, zrow, gather=False)

    dis2, y12 = pl.pallas_call(
        _deg_body,
        out_shape=(f32_2d, f32_2d),
    )(cnt.reshape(NC, R, 128), x2)

    accA = _edge_pass(y12.reshape(NPAD), ei3, zeros_pad, zrow)

    y22 = pl.pallas_call(
        _mid_body,
        out_shape=f32_2d,
        in_specs=[pl.BlockSpec(), pl.BlockSpec(), pl.BlockSpec(),
                  _SMEM_SPEC, _SMEM_SPEC, _SMEM_SPEC],
    )(accA.reshape(NC, R, 128), dis2, y12, w1r, b1r, w2r)

    accB = _edge_pass(y22.reshape(NPAD), ei3, zeros_pad, zrow)

    out2 = pl.pallas_call(
        _fin_body,
        out_shape=f32_2d,
        in_specs=[pl.BlockSpec(), pl.BlockSpec(), pl.BlockSpec(), _SMEM_SPEC],
    )(accB.reshape(NC, R, 128), dis2, y22, b2r)

    return out2.reshape(NPAD)[:N_NODES][:, None]
